# Initial kernel scaffold; baseline (speedup 1.0000x reference)
import jax
import jax.numpy as jnp
from jax.experimental import pallas as pl


def kernel(static_emb, dynamic_emb, nid, edge_index, rel_type, W_rel0, W_loop0, b0, W_rel1, W_loop1, b1, W_ih, W_hh, b_ih, b_hh):
    raise NotImplementedError("write your pallas kernel here")


# speedup vs baseline: 2.3228x; 2.3228x over previous
import functools

import jax
import jax.numpy as jnp
from jax import lax
from jax.experimental import pallas as pl
from jax.experimental.pallas import tpu as pltpu
from jax.experimental.pallas import tpu_sc as plsc

_N = 50000
_E = 800000
_D = 100
_R = 256
_DP = 128

_info = plsc.get_sparse_core_info()
_NC = _info.num_cores
_NS = _info.num_subcores
_NW = _NC * _NS


def _sc_gather(table, idx, block):
    B, = idx.shape
    V, D = table.shape
    b_per_w = B // _NW
    nblk = b_per_w // block
    mesh = plsc.VectorSubcoreMesh(core_axis_name="c", subcore_axis_name="s")

    @functools.partial(
        pl.kernel,
        mesh=mesh,
        out_type=jax.ShapeDtypeStruct((B, D), jnp.float32),
        scratch_types=[
            pltpu.VMEM((block,), jnp.int32),
            pltpu.VMEM((block, D), jnp.float32),
            pltpu.SemaphoreType.DMA,
        ],
    )
    def k(table_hbm, idx_hbm, out_hbm, idx_v, rows_v, sem):
        wid = lax.axis_index("s") * _NC + lax.axis_index("c")
        base = wid * b_per_w

        def body(i, carry):
            off = base + i * block
            pltpu.sync_copy(idx_hbm.at[pl.ds(off, block)], idx_v)
            pltpu.async_copy(table_hbm.at[idx_v], rows_v, sem).wait()
            pltpu.sync_copy(rows_v, out_hbm.at[pl.ds(off, block)])
            return carry

        lax.fori_loop(0, nblk, body, 0)

    return k(table, idx)


def _edge_mul(he, ho, a0, a1):
    E_, Dp = he.shape
    BE = 2000

    def body(he_r, ho_r, a0_r, a1_r, o_r):
        o_r[...] = he_r[...] * a0_r[...] + ho_r[...] * a1_r[...]

    return pl.pallas_call(
        body,
        grid=(E_ // BE,),
        in_specs=[pl.BlockSpec((BE, Dp), lambda i: (i, 0))] * 4,
        out_specs=pl.BlockSpec((BE, Dp), lambda i: (i, 0)),
        out_shape=jax.ShapeDtypeStruct((E_, Dp), jnp.float32),
    )(he, ho, a0, a1)


def _layer_combine(agg, inv, h, w_loop, b):
    N_, Dp = h.shape
    BN = 2000

    def body(agg_r, inv_r, h_r, w_r, b_r, o_r):
        o_r[...] = (
            agg_r[...] * inv_r[...]
            + jnp.dot(h_r[...], w_r[...], preferred_element_type=jnp.float32)
            + b_r[...]
        )

    return pl.pallas_call(
        body,
        grid=(N_ // BN,),
        in_specs=[
            pl.BlockSpec((BN, Dp), lambda i: (i, 0)),
            pl.BlockSpec((BN, 1), lambda i: (i, 0)),
            pl.BlockSpec((BN, Dp), lambda i: (i, 0)),
            pl.BlockSpec((Dp, Dp), lambda i: (0, 0)),
            pl.BlockSpec((1, Dp), lambda i: (0, 0)),
        ],
        out_specs=pl.BlockSpec((BN, Dp), lambda i: (i, 0)),
        out_shape=jax.ShapeDtypeStruct((N_, Dp), jnp.float32),
    )(agg, inv, h, w_loop, b)


def _gru(x, h0, wi, wh, bi, bh):
    N_, Dp = x.shape
    BN = 2000
    G = 3 * _DP

    def body(x_r, h0_r, wi_r, wh_r, bi_r, bh_r, o_r):
        gi = jnp.dot(x_r[...], wi_r[...], preferred_element_type=jnp.float32) + bi_r[...]
        gh = jnp.dot(h0_r[...], wh_r[...], preferred_element_type=jnp.float32) + bh_r[...]
        r = jax.nn.sigmoid(gi[:, 0:_DP] + gh[:, 0:_DP])
        z = jax.nn.sigmoid(gi[:, _DP:2 * _DP] + gh[:, _DP:2 * _DP])
        n = jnp.tanh(gi[:, 2 * _DP:] + r * gh[:, 2 * _DP:])
        o_r[...] = (1.0 - z) * n + z * h0_r[...]

    return pl.pallas_call(
        body,
        grid=(N_ // BN,),
        in_specs=[
            pl.BlockSpec((BN, Dp), lambda i: (i, 0)),
            pl.BlockSpec((BN, Dp), lambda i: (i, 0)),
            pl.BlockSpec((Dp, G), lambda i: (0, 0)),
            pl.BlockSpec((Dp, G), lambda i: (0, 0)),
            pl.BlockSpec((1, G), lambda i: (0, 0)),
            pl.BlockSpec((1, G), lambda i: (0, 0)),
        ],
        out_specs=pl.BlockSpec((BN, Dp), lambda i: (i, 0)),
        out_shape=jax.ShapeDtypeStruct((N_, Dp), jnp.float32),
    )(x, h0, wi, wh, bi, bh)


def _pad_cols(x, dp=_DP):
    return jnp.pad(x, ((0, 0), (0, dp - x.shape[1])))


def _gate_pack(w, b):
    wp = jnp.zeros((_DP, 3 * _DP), jnp.float32)
    bp = jnp.zeros((1, 3 * _DP), jnp.float32)
    for g in range(3):
        wp = wp.at[:_D, g * _DP:g * _DP + _D].set(w[g * _D:(g + 1) * _D, :].T)
        bp = bp.at[0, g * _DP:g * _DP + _D].set(b[g * _D:(g + 1) * _D])
    return wp, bp


def kernel(static_emb, dynamic_emb, nid, edge_index, rel_type,
           W_rel0, W_loop0, b0, W_rel1, W_loop1, b1,
           W_ih, W_hh, b_ih, b_hh):
    NB = W_rel0.shape[1]
    src = edge_index[0]
    dst = edge_index[1]

    deg = jax.ops.segment_sum(jnp.ones((_E,), jnp.float32), dst, num_segments=_N)
    inv = (1.0 / jnp.maximum(deg, 1.0))[:, None]

    BPAD = ((_N + 8 * _NW - 1) // (8 * _NW)) * (8 * _NW)
    nid_p = jnp.pad(nid, (0, BPAD - _N))
    st_p = _pad_cols(static_emb)
    dyn_p = _pad_cols(dynamic_emb[:, 0, :])
    h = _sc_gather(st_p, nid_p, 224)[:_N]
    h0 = _sc_gather(dyn_p, nid_p, 224)[:_N]

    def rgcn_layer(h, W_rel, W_loop, b):
        a0 = _pad_cols(W_rel[:, :, 0, :].reshape(_R, _D))
        a1 = _pad_cols(W_rel[:, :, 1, :].reshape(_R, _D))
        he = jnp.repeat(h[:, 0::2], 2, axis=1)
        ho = jnp.repeat(h[:, 1::2], 2, axis=1)
        heg = _sc_gather(he, src, 200)
        hog = _sc_gather(ho, src, 200)
        a0g = _sc_gather(a0, rel_type, 200)
        a1g = _sc_gather(a1, rel_type, 200)
        msg = _edge_mul(heg, hog, a0g, a1g)
        agg = jax.ops.segment_sum(msg, dst, num_segments=_N)
        wl = jnp.zeros((_DP, _DP), jnp.float32).at[:_D, :_D].set(W_loop)
        bp = jnp.zeros((1, _DP), jnp.float32).at[0, :_D].set(b)
        return _layer_combine(agg, inv, h, wl, bp)

    h = rgcn_layer(h, W_rel0, W_loop0, b0)
    h = rgcn_layer(h, W_rel1, W_loop1, b1)

    wi, bi = _gate_pack(W_ih, b_ih)
    wh, bh = _gate_pack(W_hh, b_hh)
    hn = _gru(h, h0, wi, wh, bi, bh)
    return hn[:, :_D][:, None, :]


# speedup vs baseline: 3.6958x; 1.5911x over previous
import functools

import jax
import jax.numpy as jnp
from jax import lax
from jax.experimental import pallas as pl
from jax.experimental.pallas import tpu as pltpu
from jax.experimental.pallas import tpu_sc as plsc

_N = 50000
_E = 800000
_D = 100
_R = 256
_DP = 128

_info = plsc.get_sparse_core_info()
_NC = _info.num_cores
_NS = _info.num_subcores
_NW = _NC * _NS


def _sc_gather(table, idx, block):
    B, = idx.shape
    V, D = table.shape
    b_per_w = B // _NW
    nblk = b_per_w // block
    mesh = plsc.VectorSubcoreMesh(core_axis_name="c", subcore_axis_name="s")

    @functools.partial(
        pl.kernel,
        mesh=mesh,
        out_type=jax.ShapeDtypeStruct((B, D), jnp.float32),
        scratch_types=[
            pltpu.VMEM((block,), jnp.int32),
            pltpu.VMEM((block, D), jnp.float32),
            pltpu.SemaphoreType.DMA,
        ],
    )
    def k(table_hbm, idx_hbm, out_hbm, idx_v, rows_v, sem):
        wid = lax.axis_index("s") * _NC + lax.axis_index("c")
        base = wid * b_per_w

        def body(i, carry):
            off = base + i * block
            pltpu.sync_copy(idx_hbm.at[pl.ds(off, block)], idx_v)
            pltpu.async_copy(table_hbm.at[idx_v], rows_v, sem).wait()
            pltpu.sync_copy(rows_v, out_hbm.at[pl.ds(off, block)])
            return carry

        lax.fori_loop(0, nblk, body, 0)

    return k(table, idx)


def _edge_msg(hs, rel2d, se, so, a0, a1):
    E_, Dp = hs.shape
    BE = 2000

    def body(hs_r, rel_r, se_r, so_r, a0_r, a1_r, o_r):
        oh = (rel_r[...] == lax.broadcasted_iota(jnp.int32, (BE, _R), 1))
        oh = oh.astype(jnp.float32)
        w0 = jnp.dot(oh, a0_r[...], preferred_element_type=jnp.float32)
        w1 = jnp.dot(oh, a1_r[...], preferred_element_type=jnp.float32)
        he = jnp.dot(hs_r[...], se_r[...], preferred_element_type=jnp.float32)
        ho = jnp.dot(hs_r[...], so_r[...], preferred_element_type=jnp.float32)
        o_r[...] = he * w0 + ho * w1

    return pl.pallas_call(
        body,
        grid=(E_ // BE,),
        in_specs=[
            pl.BlockSpec((BE, Dp), lambda i: (i, 0)),
            pl.BlockSpec((BE, 1), lambda i: (i, 0)),
            pl.BlockSpec((Dp, Dp), lambda i: (0, 0)),
            pl.BlockSpec((Dp, Dp), lambda i: (0, 0)),
            pl.BlockSpec((_R, Dp), lambda i: (0, 0)),
            pl.BlockSpec((_R, Dp), lambda i: (0, 0)),
        ],
        out_specs=pl.BlockSpec((BE, Dp), lambda i: (i, 0)),
        out_shape=jax.ShapeDtypeStruct((E_, Dp), jnp.float32),
    )(hs, rel2d, se, so, a0, a1)


def _layer_combine(agg, inv, h, w_loop, b):
    N_, Dp = h.shape
    BN = 2000

    def body(agg_r, inv_r, h_r, w_r, b_r, o_r):
        o_r[...] = (
            agg_r[...] * inv_r[...]
            + jnp.dot(h_r[...], w_r[...], preferred_element_type=jnp.float32)
            + b_r[...]
        )

    return pl.pallas_call(
        body,
        grid=(N_ // BN,),
        in_specs=[
            pl.BlockSpec((BN, Dp), lambda i: (i, 0)),
            pl.BlockSpec((BN, 1), lambda i: (i, 0)),
            pl.BlockSpec((BN, Dp), lambda i: (i, 0)),
            pl.BlockSpec((Dp, Dp), lambda i: (0, 0)),
            pl.BlockSpec((1, Dp), lambda i: (0, 0)),
        ],
        out_specs=pl.BlockSpec((BN, Dp), lambda i: (i, 0)),
        out_shape=jax.ShapeDtypeStruct((N_, Dp), jnp.float32),
    )(agg, inv, h, w_loop, b)


def _gru(x, h0, wi, wh, bi, bh):
    N_, Dp = x.shape
    BN = 2000
    G = 3 * _DP

    def body(x_r, h0_r, wi_r, wh_r, bi_r, bh_r, o_r):
        gi = jnp.dot(x_r[...], wi_r[...], preferred_element_type=jnp.float32) + bi_r[...]
        gh = jnp.dot(h0_r[...], wh_r[...], preferred_element_type=jnp.float32) + bh_r[...]
        r = jax.nn.sigmoid(gi[:, 0:_DP] + gh[:, 0:_DP])
        z = jax.nn.sigmoid(gi[:, _DP:2 * _DP] + gh[:, _DP:2 * _DP])
        n = jnp.tanh(gi[:, 2 * _DP:] + r * gh[:, 2 * _DP:])
        o_r[...] = (1.0 - z) * n + z * h0_r[...]

    return pl.pallas_call(
        body,
        grid=(N_ // BN,),
        in_specs=[
            pl.BlockSpec((BN, Dp), lambda i: (i, 0)),
            pl.BlockSpec((BN, Dp), lambda i: (i, 0)),
            pl.BlockSpec((Dp, G), lambda i: (0, 0)),
            pl.BlockSpec((Dp, G), lambda i: (0, 0)),
            pl.BlockSpec((1, G), lambda i: (0, 0)),
            pl.BlockSpec((1, G), lambda i: (0, 0)),
        ],
        out_specs=pl.BlockSpec((BN, Dp), lambda i: (i, 0)),
        out_shape=jax.ShapeDtypeStruct((N_, Dp), jnp.float32),
    )(x, h0, wi, wh, bi, bh)


def _pad_cols(x, dp=_DP):
    return jnp.pad(x, ((0, 0), (0, dp - x.shape[1])))


def _gate_pack(w, b):
    wp = jnp.zeros((_DP, 3 * _DP), jnp.float32)
    bp = jnp.zeros((1, 3 * _DP), jnp.float32)
    for g in range(3):
        wp = wp.at[:_D, g * _DP:g * _DP + _D].set(w[g * _D:(g + 1) * _D, :].T)
        bp = bp.at[0, g * _DP:g * _DP + _D].set(b[g * _D:(g + 1) * _D])
    return wp, bp


def kernel(static_emb, dynamic_emb, nid, edge_index, rel_type,
           W_rel0, W_loop0, b0, W_rel1, W_loop1, b1,
           W_ih, W_hh, b_ih, b_hh):
    NB = W_rel0.shape[1]
    src = edge_index[0]
    dst = edge_index[1]

    deg = jax.ops.segment_sum(jnp.ones((_E,), jnp.float32), dst, num_segments=_N)
    inv = (1.0 / jnp.maximum(deg, 1.0))[:, None]

    BPAD = ((_N + 8 * _NW - 1) // (8 * _NW)) * (8 * _NW)
    nid_p = jnp.pad(nid, (0, BPAD - _N))
    st_p = _pad_cols(static_emb)
    dyn_p = _pad_cols(dynamic_emb[:, 0, :])
    h = _sc_gather(st_p, nid_p, 224)[:_N]
    h0 = _sc_gather(dyn_p, nid_p, 224)[:_N]

    cols = jnp.arange(_DP)
    se = jnp.zeros((_DP, _DP), jnp.float32).at[cols - cols % 2, cols].set(1.0)
    so = jnp.zeros((_DP, _DP), jnp.float32).at[cols - cols % 2 + 1, cols].set(1.0)
    rel2d = rel_type[:, None]

    def rgcn_layer(h, W_rel, W_loop, b):
        a0 = _pad_cols(W_rel[:, :, 0, :].reshape(_R, _D))
        a1 = _pad_cols(W_rel[:, :, 1, :].reshape(_R, _D))
        hsg = _sc_gather(h, src, 200)
        msg = _edge_msg(hsg, rel2d, se, so, a0, a1)
        agg = jax.ops.segment_sum(msg, dst, num_segments=_N)
        wl = jnp.zeros((_DP, _DP), jnp.float32).at[:_D, :_D].set(W_loop)
        bp = jnp.zeros((1, _DP), jnp.float32).at[0, :_D].set(b)
        return _layer_combine(agg, inv, h, wl, bp)

    h = rgcn_layer(h, W_rel0, W_loop0, b0)
    h = rgcn_layer(h, W_rel1, W_loop1, b1)

    wi, bi = _gate_pack(W_ih, b_ih)
    wh, bh = _gate_pack(W_hh, b_hh)
    hn = _gru(h, h0, wi, wh, bi, bh)
    return hn[:, :_D][:, None, :]


# speedup vs baseline: 3.8981x; 1.0547x over previous
import functools

import jax
import jax.numpy as jnp
from jax import lax
from jax.experimental import pallas as pl
from jax.experimental.pallas import tpu as pltpu
from jax.experimental.pallas import tpu_sc as plsc

_N = 50000
_E = 800000
_D = 100
_R = 256
_DP = 128

_info = plsc.get_sparse_core_info()
_NC = _info.num_cores
_NS = _info.num_subcores
_NW = _NC * _NS


def _sc_gather(table, idx, block):
    B, = idx.shape
    V, D = table.shape
    b_per_w = B // _NW
    nblk = b_per_w // block
    mesh = plsc.VectorSubcoreMesh(core_axis_name="c", subcore_axis_name="s")

    @functools.partial(
        pl.kernel,
        mesh=mesh,
        out_type=jax.ShapeDtypeStruct((B, D), jnp.float32),
        scratch_types=[
            pltpu.VMEM((block,), jnp.int32),
            pltpu.VMEM((block, D), jnp.float32),
            pltpu.SemaphoreType.DMA,
        ],
    )
    def k(table_hbm, idx_hbm, out_hbm, idx_v, rows_v, sem):
        wid = lax.axis_index("s") * _NC + lax.axis_index("c")
        base = wid * b_per_w

        def body(i, carry):
            off = base + i * block
            pltpu.sync_copy(idx_hbm.at[pl.ds(off, block)], idx_v)
            pltpu.async_copy(table_hbm.at[idx_v], rows_v, sem).wait()
            pltpu.sync_copy(rows_v, out_hbm.at[pl.ds(off, block)])
            return carry

        lax.fori_loop(0, nblk, body, 0)

    return k(table, idx)


def _edge_msg(hs, rel2d, se, so, a0, a1):
    E_, Dp = hs.shape
    BE = 8000

    def body(hs_r, rel_r, se_r, so_r, a0_r, a1_r, o_r):
        oh = (rel_r[...] == lax.broadcasted_iota(jnp.int32, (BE, _R), 1))
        oh = oh.astype(jnp.float32)
        w0 = jnp.dot(oh, a0_r[...], preferred_element_type=jnp.float32)
        w1 = jnp.dot(oh, a1_r[...], preferred_element_type=jnp.float32)
        he = jnp.dot(hs_r[...], se_r[...], preferred_element_type=jnp.float32)
        ho = jnp.dot(hs_r[...], so_r[...], preferred_element_type=jnp.float32)
        o_r[...] = he * w0 + ho * w1

    return pl.pallas_call(
        body,
        grid=(E_ // BE,),
        in_specs=[
            pl.BlockSpec((BE, Dp), lambda i: (i, 0)),
            pl.BlockSpec((BE, 1), lambda i: (i, 0)),
            pl.BlockSpec((Dp, Dp), lambda i: (0, 0)),
            pl.BlockSpec((Dp, Dp), lambda i: (0, 0)),
            pl.BlockSpec((_R, Dp), lambda i: (0, 0)),
            pl.BlockSpec((_R, Dp), lambda i: (0, 0)),
        ],
        out_specs=pl.BlockSpec((BE, Dp), lambda i: (i, 0)),
        out_shape=jax.ShapeDtypeStruct((E_, Dp), jnp.float32),
    )(hs, rel2d, se, so, a0, a1)


def _layer_combine(agg, inv, h, w_loop, b):
    N_, Dp = h.shape
    BN = 2000

    def body(agg_r, inv_r, h_r, w_r, b_r, o_r):
        o_r[...] = (
            agg_r[...] * inv_r[...]
            + jnp.dot(h_r[...], w_r[...], preferred_element_type=jnp.float32)
            + b_r[...]
        )

    return pl.pallas_call(
        body,
        grid=(N_ // BN,),
        in_specs=[
            pl.BlockSpec((BN, Dp), lambda i: (i, 0)),
            pl.BlockSpec((BN, 1), lambda i: (i, 0)),
            pl.BlockSpec((BN, Dp), lambda i: (i, 0)),
            pl.BlockSpec((Dp, Dp), lambda i: (0, 0)),
            pl.BlockSpec((1, Dp), lambda i: (0, 0)),
        ],
        out_specs=pl.BlockSpec((BN, Dp), lambda i: (i, 0)),
        out_shape=jax.ShapeDtypeStruct((N_, Dp), jnp.float32),
    )(agg, inv, h, w_loop, b)


def _gru(x, h0, wi, wh, bi, bh):
    N_, Dp = x.shape
    BN = 2000
    G = 3 * _DP

    def body(x_r, h0_r, wi_r, wh_r, bi_r, bh_r, o_r):
        gi = jnp.dot(x_r[...], wi_r[...], preferred_element_type=jnp.float32) + bi_r[...]
        gh = jnp.dot(h0_r[...], wh_r[...], preferred_element_type=jnp.float32) + bh_r[...]
        r = jax.nn.sigmoid(gi[:, 0:_DP] + gh[:, 0:_DP])
        z = jax.nn.sigmoid(gi[:, _DP:2 * _DP] + gh[:, _DP:2 * _DP])
        n = jnp.tanh(gi[:, 2 * _DP:] + r * gh[:, 2 * _DP:])
        o_r[...] = (1.0 - z) * n + z * h0_r[...]

    return pl.pallas_call(
        body,
        grid=(N_ // BN,),
        in_specs=[
            pl.BlockSpec((BN, Dp), lambda i: (i, 0)),
            pl.BlockSpec((BN, Dp), lambda i: (i, 0)),
            pl.BlockSpec((Dp, G), lambda i: (0, 0)),
            pl.BlockSpec((Dp, G), lambda i: (0, 0)),
            pl.BlockSpec((1, G), lambda i: (0, 0)),
            pl.BlockSpec((1, G), lambda i: (0, 0)),
        ],
        out_specs=pl.BlockSpec((BN, Dp), lambda i: (i, 0)),
        out_shape=jax.ShapeDtypeStruct((N_, Dp), jnp.float32),
    )(x, h0, wi, wh, bi, bh)


def _pad_cols(x, dp=_DP):
    return jnp.pad(x, ((0, 0), (0, dp - x.shape[1])))


def _gate_pack(w, b):
    wp = jnp.zeros((_DP, 3 * _DP), jnp.float32)
    bp = jnp.zeros((1, 3 * _DP), jnp.float32)
    for g in range(3):
        wp = wp.at[:_D, g * _DP:g * _DP + _D].set(w[g * _D:(g + 1) * _D, :].T)
        bp = bp.at[0, g * _DP:g * _DP + _D].set(b[g * _D:(g + 1) * _D])
    return wp, bp


def kernel(static_emb, dynamic_emb, nid, edge_index, rel_type,
           W_rel0, W_loop0, b0, W_rel1, W_loop1, b1,
           W_ih, W_hh, b_ih, b_hh):
    NB = W_rel0.shape[1]
    src = edge_index[0]
    dst = edge_index[1]

    deg = jax.ops.segment_sum(jnp.ones((_E,), jnp.float32), dst, num_segments=_N)
    inv = (1.0 / jnp.maximum(deg, 1.0))[:, None]

    BPAD = ((_N + 8 * _NW - 1) // (8 * _NW)) * (8 * _NW)
    nid_p = jnp.pad(nid, (0, BPAD - _N))
    st_p = _pad_cols(static_emb)
    dyn_p = _pad_cols(dynamic_emb[:, 0, :])
    h = _sc_gather(st_p, nid_p, 224)[:_N]
    h0 = _sc_gather(dyn_p, nid_p, 224)[:_N]

    cols = jnp.arange(_DP)
    se = jnp.zeros((_DP, _DP), jnp.float32).at[cols - cols % 2, cols].set(1.0)
    so = jnp.zeros((_DP, _DP), jnp.float32).at[cols - cols % 2 + 1, cols].set(1.0)
    rel2d = rel_type[:, None]

    def rgcn_layer(h, W_rel, W_loop, b):
        a0 = _pad_cols(W_rel[:, :, 0, :].reshape(_R, _D))
        a1 = _pad_cols(W_rel[:, :, 1, :].reshape(_R, _D))
        hsg = _sc_gather(h, src, 200)
        msg = _edge_msg(hsg, rel2d, se, so, a0, a1)
        agg = jax.ops.segment_sum(msg, dst, num_segments=_N)
        wl = jnp.zeros((_DP, _DP), jnp.float32).at[:_D, :_D].set(W_loop)
        bp = jnp.zeros((1, _DP), jnp.float32).at[0, :_D].set(b)
        return _layer_combine(agg, inv, h, wl, bp)

    h = rgcn_layer(h, W_rel0, W_loop0, b0)
    h = rgcn_layer(h, W_rel1, W_loop1, b1)

    wi, bi = _gate_pack(W_ih, b_ih)
    wh, bh = _gate_pack(W_hh, b_hh)
    hn = _gru(h, h0, wi, wh, bi, bh)
    return hn[:, :_D][:, None, :]
